# baseline (speedup 1.0000x reference)
import functools

import jax
import jax.numpy as jnp
from jax import lax
from jax.experimental import pallas as pl
from jax.experimental.pallas import tpu as pltpu
from jax.experimental.pallas import tpu_sc as plsc

NB = 4


def _make_gather(B, H, V, D, NC, NS):
    NW = NC * NS
    assert B % NW == 0
    T = B // NW
    assert T % NB == 0 and T // NB >= 2
    ngroups = T // NB

    mesh = plsc.VectorSubcoreMesh(core_axis_name="c", subcore_axis_name="s")

    @functools.partial(
        pl.kernel,
        mesh=mesh,
        out_type=jax.ShapeDtypeStruct((B, H, D), jnp.float32),
        scratch_types=[
            pltpu.VMEM((NB, H), jnp.int32),
            pltpu.VMEM((NB, H, D), jnp.float32),
            pltpu.SemaphoreType.DMA((NB,)),
            pltpu.SemaphoreType.DMA((NB,)),
            pltpu.SemaphoreType.DMA((NB,)),
        ],
        compiler_params=pltpu.CompilerParams(use_tc_tiling_on_sc=False),
    )
    def gather(ids_hbm, table_hbm, out_hbm, idx_v, rows_v, s_idx, s_gth, s_out):
        wid = lax.axis_index("s") * NC + lax.axis_index("c")
        base = wid * T

        def cp_idx(g, b):
            off = (base + g) * H
            return pltpu.make_async_copy(
                ids_hbm.at[pl.ds(off, H)], idx_v.at[b], s_idx.at[b])

        def cp_gth(b):
            return pltpu.make_async_copy(
                table_hbm.at[idx_v.at[b]], rows_v.at[b], s_gth.at[b])

        def cp_out(g, b):
            return pltpu.make_async_copy(
                rows_v.at[b], out_hbm.at[base + g], s_out.at[b])

        def body(g, b, do_next, do_out_wait, do_prefetch):
            bn = (b + 1) % NB
            bp = (b + 2) % NB
            if do_next:
                cp_idx(g + 1, bn).wait()
                if do_out_wait:
                    cp_out(g + 1 - NB, bn).wait()
                cp_gth(bn).start()
            cp_gth(b).wait()
            cp_out(g, b).start()
            if do_prefetch:
                cp_idx(g + 2, bp).start()

        cp_idx(0, 0).start()
        cp_idx(1, 1).start()
        cp_idx(0, 0).wait()
        cp_gth(0).start()

        for b in range(NB):
            body(b, b, True, b + 1 >= NB, True)

        @pl.loop(1, ngroups - 1)
        def _(t):
            gbase = t * NB
            for b in range(NB):
                body(gbase + b, b, True, True, True)

        gl = (ngroups - 1) * NB
        for b in range(NB):
            g = gl + b
            body(g, b, g + 1 < T, True, g + 2 < T)

        for b in range(NB):
            cp_out(gl + b, b).wait()

    return gather


def kernel(doctor_ids, embedding_table):
    B, H = doctor_ids.shape
    V, D = embedding_table.shape
    flat_idx = doctor_ids.reshape(B * H).astype(jnp.int32)
    info = plsc.get_sparse_core_info()
    gather = _make_gather(B, H, V, D, info.num_cores, info.num_subcores)
    return gather(flat_idx, embedding_table)


# speedup vs baseline: 1.6442x; 1.6442x over previous
import functools

import jax
import jax.numpy as jnp
from jax import lax
from jax.experimental import pallas as pl
from jax.experimental.pallas import tpu as pltpu
from jax.experimental.pallas import tpu_sc as plsc

NB = 4


def _make_gather(B, H, V, D, NC, NS):
    NW = NC * NS
    assert B % NW == 0
    T = B // NW
    assert T % NB == 0 and T // NB >= 2
    ngroups = T // NB

    mesh = plsc.VectorSubcoreMesh(core_axis_name="c", subcore_axis_name="s")

    @functools.partial(
        pl.kernel,
        mesh=mesh,
        out_type=jax.ShapeDtypeStruct((B * H, 2 * D), jnp.float32),
        scratch_types=[
            pltpu.VMEM((NB, H), jnp.int32),
            pltpu.VMEM((NB, H, D), jnp.float32),
            pltpu.SemaphoreType.DMA((NB,)),
            pltpu.SemaphoreType.DMA((NB,)),
            pltpu.SemaphoreType.DMA((NB,)),
        ],
        compiler_params=pltpu.CompilerParams(use_tc_tiling_on_sc=False),
    )
    def gather(ids_hbm, table_hbm, out_hbm, idx_v, rows_v, s_idx, s_gth, s_out):
        wid = lax.axis_index("s") * NC + lax.axis_index("c")
        base = wid * T

        def cp_idx(g, b):
            off = (base + g) * H
            return pltpu.make_async_copy(
                ids_hbm.at[pl.ds(off, H)], idx_v.at[b], s_idx.at[b])

        def cp_gth(b):
            return pltpu.make_async_copy(
                table_hbm.at[idx_v.at[b]], rows_v.at[b], s_gth.at[b])

        def cp_out(g, b):
            off = (base + g) * H
            return pltpu.make_async_copy(
                rows_v.at[b],
                out_hbm.at[pl.ds(off, H), pl.ds(0, D)], s_out.at[b])

        def body(g, b, do_next, do_out_wait, do_prefetch):
            bn = (b + 1) % NB
            bp = (b + 2) % NB
            if do_next:
                cp_idx(g + 1, bn).wait()
                if do_out_wait:
                    cp_out(g + 1 - NB, bn).wait()
                cp_gth(bn).start()
            cp_gth(b).wait()
            cp_out(g, b).start()
            if do_prefetch:
                cp_idx(g + 2, bp).start()

        cp_idx(0, 0).start()
        cp_idx(1, 1).start()
        cp_idx(0, 0).wait()
        cp_gth(0).start()

        for b in range(NB):
            body(b, b, True, b + 1 >= NB, True)

        @pl.loop(1, ngroups - 1)
        def _(t):
            gbase = t * NB
            for b in range(NB):
                body(gbase + b, b, True, True, True)

        gl = (ngroups - 1) * NB
        for b in range(NB):
            g = gl + b
            body(g, b, g + 1 < T, True, g + 2 < T)

        for b in range(NB):
            cp_out(gl + b, b).wait()

    return gather


def kernel(doctor_ids, embedding_table):
    B, H = doctor_ids.shape
    V, D = embedding_table.shape
    flat_idx = doctor_ids.reshape(B * H).astype(jnp.int32)
    info = plsc.get_sparse_core_info()
    gather = _make_gather(B, H, V, D, info.num_cores, info.num_subcores)
    out = gather(flat_idx, embedding_table)
    return out[:, :D].reshape(B, H, D)


# speedup vs baseline: 1.6723x; 1.0171x over previous
import functools

import jax
import jax.numpy as jnp
from jax import lax
from jax.experimental import pallas as pl
from jax.experimental.pallas import tpu as pltpu
from jax.experimental.pallas import tpu_sc as plsc

NB = 4
CHUNK = 400


def _make_gather(N, V, D, NC, NS):
    NW = NC * NS
    per_w = N // NW
    assert per_w % CHUNK == 0 and (per_w // CHUNK) % NB == 0
    T = per_w // CHUNK
    ngroups = T // NB

    mesh = plsc.VectorSubcoreMesh(core_axis_name="c", subcore_axis_name="s")

    @functools.partial(
        pl.kernel,
        mesh=mesh,
        out_type=jax.ShapeDtypeStruct((N, 2 * D), jnp.float32),
        scratch_types=[
            pltpu.VMEM((NB, CHUNK), jnp.int32),
            pltpu.VMEM((NB, CHUNK, D), jnp.float32),
            pltpu.SemaphoreType.DMA((NB,)),
            pltpu.SemaphoreType.DMA((NB,)),
            pltpu.SemaphoreType.DMA((NB,)),
        ],
        compiler_params=pltpu.CompilerParams(use_tc_tiling_on_sc=False),
    )
    def gather(ids_hbm, table_hbm, out_hbm, idx_v, rows_v, s_idx, s_gth, s_out):
        wid = lax.axis_index("s") * NC + lax.axis_index("c")
        base = wid * per_w

        def cp_idx(g, b):
            off = base + g * CHUNK
            return pltpu.make_async_copy(
                ids_hbm.at[pl.ds(off, CHUNK)], idx_v.at[b], s_idx.at[b])

        def cp_gth(b):
            return pltpu.make_async_copy(
                table_hbm.at[idx_v.at[b]], rows_v.at[b], s_gth.at[b])

        def cp_out(g, b):
            off = base + g * CHUNK
            return pltpu.make_async_copy(
                rows_v.at[b],
                out_hbm.at[pl.ds(off, CHUNK), pl.ds(0, D)], s_out.at[b])

        def body(g, b, do_next, do_out_wait, do_prefetch):
            bn = (b + 1) % NB
            bp = (b + 2) % NB
            if do_next:
                cp_idx(g + 1, bn).wait()
                if do_out_wait:
                    cp_out(g + 1 - NB, bn).wait()
                cp_gth(bn).start()
            cp_gth(b).wait()
            cp_out(g, b).start()
            if do_prefetch:
                cp_idx(g + 2, bp).start()

        cp_idx(0, 0).start()
        cp_idx(1, 1).start()
        cp_idx(0, 0).wait()
        cp_gth(0).start()

        for b in range(NB):
            body(b, b, True, b + 1 >= NB, True)

        @pl.loop(1, ngroups - 1)
        def _(t):
            gbase = t * NB
            for b in range(NB):
                body(gbase + b, b, True, True, True)

        gl = (ngroups - 1) * NB
        for b in range(NB):
            g = gl + b
            body(g, b, g + 1 < T, True, g + 2 < T)

        for b in range(NB):
            cp_out(gl + b, b).wait()

    return gather


def kernel(doctor_ids, embedding_table):
    B, H = doctor_ids.shape
    V, D = embedding_table.shape
    N = B * H
    flat_idx = doctor_ids.reshape(N).astype(jnp.int32)
    info = plsc.get_sparse_core_info()
    gather = _make_gather(N, V, D, info.num_cores, info.num_subcores)
    out = gather(flat_idx, embedding_table)
    return out[:, :D].reshape(B, H, D)
